# baseline (speedup 1.0000x reference)
import jax
import jax.numpy as jnp
from jax.experimental import pallas as pl
from jax.experimental.pallas import tpu as pltpu

_CLASS_NUM = 10
_ANCHOR_NUM = 3
_SCALE = 4.0
_G = 0.5
_BBOX_W = 1.0
_OBJ_W = 20.0
_CLS_W = 10.0


def _prep(p_shape, targets, anchor):
    K, J, I = int(p_shape[2]), int(p_shape[3]), int(p_shape[4])
    B, M = targets.shape[0], targets.shape[1]
    bs = jnp.broadcast_to(jnp.arange(B, dtype=jnp.float32)[:, None, None], (B, M, 1))
    bs_targets = jnp.concatenate([bs, targets], axis=-1).reshape(B * M, -1)
    mask = (targets[..., 4] > 0.5).reshape(-1)
    tn = bs_targets.shape[0]
    ai = jnp.broadcast_to(jnp.arange(_ANCHOR_NUM, dtype=jnp.float32)[:, None], (_ANCHOR_NUM, tn))
    t = jnp.concatenate(
        [jnp.broadcast_to(bs_targets[None], (_ANCHOR_NUM, tn, bs_targets.shape[1])), ai[..., None]],
        axis=2)
    off = jnp.array([[0, 0, 0], [1, 0, 0], [0, 1, 0], [0, 0, 1],
                     [-1, 0, 0], [0, -1, 0], [0, 0, -1]], dtype=jnp.float32) * _G
    anchor_norm = anchor[0] / _SCALE
    t = t.at[..., 1:5].set(t[..., 1:5] / _SCALE)
    r = (t[..., 4] / anchor_norm)[..., None]
    j = jnp.max(jnp.maximum(r, 1.0 / r), axis=2) < 4.0
    valid1 = (mask[None] & j).reshape(-1)
    t = t.reshape(_ANCHOR_NUM * tn, -1)
    gxyz0 = t[:, 1:4]
    gain_v = jnp.array([K, J, I], dtype=jnp.float32)
    gxyz_i = gain_v - gxyz0
    a, b, c = ((gxyz0 % 1 < _G) & (gxyz0 > 1)).T
    d, e, f = ((gxyz_i % 1 < _G) & (gxyz_i > 1)).T
    fm = jnp.stack([jnp.ones_like(a), a, b, c, d, e, f])
    valid = (valid1[None] & fm).reshape(-1)
    t = jnp.broadcast_to(t[None], (7,) + t.shape).reshape(7 * _ANCHOR_NUM * tn, -1)
    offsets = (jnp.zeros_like(gxyz0)[None] + off[:, None, :]).reshape(7 * _ANCHOR_NUM * tn, 3)
    b_idx = t[:, 0].astype(jnp.int32)
    a_idx = t[:, -1].astype(jnp.int32)
    tcls = t[:, 6:-1]
    gxyz = t[:, 1:4]
    gr_ = t[:, 4]
    gijk = (gxyz - offsets).astype(jnp.int32)
    gi = jnp.clip(gijk[:, 0], 0, I - 1)
    gj = jnp.clip(gijk[:, 1], 0, J - 1)
    gk = jnp.clip(gijk[:, 2], 0, K - 1)
    gijk_cl = jnp.stack([gi, gj, gk], axis=1).astype(jnp.float32)
    tbox = jnp.concatenate([gxyz - gijk_cl, gr_[:, None]], axis=1)
    anch = anchor_norm[a_idx]
    return (b_idx, a_idx, gk, gj, gi), tbox, anch, tcls, valid


def _obj_kernel(x_ref, t_ref, o_ref):
    i = pl.program_id(0)
    x = x_ref[...]
    t = t_ref[...]
    s = jnp.sum(jnp.maximum(x, 0.0) - x * t + jnp.log1p(jnp.exp(-jnp.abs(x))))

    @pl.when(i == 0)
    def _():
        o_ref[0, 0] = s

    @pl.when(i != 0)
    def _():
        o_ref[0, 0] = o_ref[0, 0] + s


def _match_kernel(pred_ref, tbox_ref, anch_ref, tcls_ref, valid_ref, o_ref):
    pred = pred_ref[...]
    tbox = tbox_ref[...]
    anch = anch_ref[...]
    tcls = tcls_ref[...]
    v = valid_ref[...]

    def sig(z):
        return 1.0 / (1.0 + jnp.exp(-z))

    eps = 1e-7
    x1 = sig(pred[0:1, :]) * 2.0 - 0.5
    y1 = sig(pred[1:2, :]) * 2.0 - 0.5
    z1 = sig(pred[2:3, :]) * 2.0 - 0.5
    r1 = (sig(pred[3:4, :]) * 2.0) ** 2 * anch
    x2 = tbox[0:1, :]
    y2 = tbox[1:2, :]
    z2 = tbox[2:3, :]
    r2 = tbox[3:4, :]
    iw = jnp.clip(jnp.minimum(x1 + r1, x2 + r2) - jnp.maximum(x1 - r1, x2 - r2), 0.0, None)
    ih = jnp.clip(jnp.minimum(y1 + r1, y2 + r2) - jnp.maximum(y1 - r1, y2 - r2), 0.0, None)
    idp = jnp.clip(jnp.minimum(z1 + r1, z2 + r2) - jnp.maximum(z1 - r1, z2 - r2), 0.0, None)
    inter = iw * ih * idp
    vol1 = (2.0 * r1) ** 3
    vol2 = (2.0 * r2) ** 3
    union = vol1 + vol2 - inter + eps
    iou = inter / union
    cw = jnp.maximum(x1 + r1, x2 + r2) - jnp.minimum(x1 - r1, x2 - r2)
    ch = jnp.maximum(y1 + r1, y2 + r2) - jnp.minimum(y1 - r1, y2 - r2)
    cd = jnp.maximum(z1 + r1, z2 + r2) - jnp.minimum(z1 - r1, z2 - r2)
    c2 = cw ** 2 + ch ** 2 + cd ** 2 + eps
    rho2 = (x1 - x2) ** 2 + (y1 - y2) ** 2 + (z1 - z2) ** 2
    dr2 = (2.0 * r1 - 2.0 * r2) ** 2
    eiou = iou - rho2 / c2 - dr2 / (cw ** 2 + eps) - dr2 / (ch ** 2 + eps) - dr2 / (cd ** 2 + eps)

    pc = pred[5:15, :]
    elem = jnp.maximum(pc, 0.0) - pc * tcls + jnp.log1p(jnp.exp(-jnp.abs(pc)))

    o_ref[0, 0] = jnp.sum(v)
    o_ref[0, 1] = jnp.sum(v * (1.0 - eiou))
    o_ref[0, 2] = jnp.sum(v * elem)


def kernel(p, targets, anchor):
    B, A, K, J, I, C = p.shape
    indices, tbox, anch, tcls, valid = _prep(p.shape, targets, anchor)
    b_idx, a_idx, gk, gj, gi = indices

    flat = (((b_idx * A + a_idx) * K + gk) * J + gj) * I + gi
    p2 = p.reshape(B * A * K * J * I, C)
    pred = jnp.take(p2, flat, axis=0)

    bs_sc = jnp.where(valid, b_idx, jnp.int32(B))
    flat_sc = (((bs_sc * A + a_idx) * K + gk) * J + gj) * I + gi
    tobj = jnp.zeros((B * A * K * J * I,), jnp.float32).at[flat_sc].set(1.0, mode='drop')

    ncell = B * A * K * J * I
    x2d = p[..., 4].reshape(2592, 1024)
    t2d = tobj.reshape(2592, 1024)
    obj_sum = pl.pallas_call(
        _obj_kernel,
        grid=(9,),
        in_specs=[
            pl.BlockSpec((288, 1024), lambda i: (i, 0)),
            pl.BlockSpec((288, 1024), lambda i: (i, 0)),
        ],
        out_specs=pl.BlockSpec((1, 1), lambda i: (0, 0), memory_space=pltpu.SMEM),
        out_shape=jax.ShapeDtypeStruct((1, 1), jnp.float32),
    )(x2d, t2d)

    validf = valid.astype(jnp.float32)[None, :]
    sums = pl.pallas_call(
        _match_kernel,
        out_shape=jax.ShapeDtypeStruct((1, 3), jnp.float32),
        out_specs=pl.BlockSpec(memory_space=pltpu.SMEM),
    )(pred.T, tbox.T, anch.T, tcls.T, validf)

    cnt = sums[0, 0]
    denom = jnp.maximum(cnt, 1.0)
    loss_bbox = (sums[0, 1] / denom).reshape(1) * _BBOX_W
    loss_cls = (sums[0, 2] / (denom * _CLASS_NUM)).reshape(1) * _CLS_W
    loss_obj = (obj_sum[0, 0] / ncell).reshape(1) * _OBJ_W
    total = (loss_bbox + loss_obj + loss_cls) * B
    return (total, loss_obj, loss_cls)


# speedup vs baseline: 6.0347x; 6.0347x over previous
import jax
import jax.numpy as jnp
from jax.experimental import pallas as pl
from jax.experimental.pallas import tpu as pltpu

_CLASS_NUM = 10
_ANCHOR_NUM = 3
_SCALE = 4.0
_G = 0.5
_BBOX_W = 1.0
_OBJ_W = 20.0
_CLS_W = 10.0


def _prep(p_shape, targets, anchor):
    K, J, I = int(p_shape[2]), int(p_shape[3]), int(p_shape[4])
    B, M = targets.shape[0], targets.shape[1]
    bs = jnp.broadcast_to(jnp.arange(B, dtype=jnp.float32)[:, None, None], (B, M, 1))
    bs_targets = jnp.concatenate([bs, targets], axis=-1).reshape(B * M, -1)
    mask = (targets[..., 4] > 0.5).reshape(-1)
    tn = bs_targets.shape[0]
    ai = jnp.broadcast_to(jnp.arange(_ANCHOR_NUM, dtype=jnp.float32)[:, None], (_ANCHOR_NUM, tn))
    t = jnp.concatenate(
        [jnp.broadcast_to(bs_targets[None], (_ANCHOR_NUM, tn, bs_targets.shape[1])), ai[..., None]],
        axis=2)
    off = jnp.array([[0, 0, 0], [1, 0, 0], [0, 1, 0], [0, 0, 1],
                     [-1, 0, 0], [0, -1, 0], [0, 0, -1]], dtype=jnp.float32) * _G
    anchor_norm = anchor[0] / _SCALE
    t = t.at[..., 1:5].set(t[..., 1:5] / _SCALE)
    r = (t[..., 4] / anchor_norm)[..., None]
    j = jnp.max(jnp.maximum(r, 1.0 / r), axis=2) < 4.0
    valid1 = (mask[None] & j).reshape(-1)
    t = t.reshape(_ANCHOR_NUM * tn, -1)
    gxyz0 = t[:, 1:4]
    gain_v = jnp.array([K, J, I], dtype=jnp.float32)
    gxyz_i = gain_v - gxyz0
    a, b, c = ((gxyz0 % 1 < _G) & (gxyz0 > 1)).T
    d, e, f = ((gxyz_i % 1 < _G) & (gxyz_i > 1)).T
    fm = jnp.stack([jnp.ones_like(a), a, b, c, d, e, f])
    valid = (valid1[None] & fm).reshape(-1)
    t = jnp.broadcast_to(t[None], (7,) + t.shape).reshape(7 * _ANCHOR_NUM * tn, -1)
    offsets = (jnp.zeros_like(gxyz0)[None] + off[:, None, :]).reshape(7 * _ANCHOR_NUM * tn, 3)
    b_idx = t[:, 0].astype(jnp.int32)
    a_idx = t[:, -1].astype(jnp.int32)
    tcls = t[:, 6:-1]
    gxyz = t[:, 1:4]
    gr_ = t[:, 4]
    gijk = (gxyz - offsets).astype(jnp.int32)
    gi = jnp.clip(gijk[:, 0], 0, I - 1)
    gj = jnp.clip(gijk[:, 1], 0, J - 1)
    gk = jnp.clip(gijk[:, 2], 0, K - 1)
    gijk_cl = jnp.stack([gi, gj, gk], axis=1).astype(jnp.float32)
    tbox = jnp.concatenate([gxyz - gijk_cl, gr_[:, None]], axis=1)
    anch = anchor_norm[a_idx]
    return (b_idx, a_idx, gk, gj, gi), tbox, anch, tcls, valid


def _obj_kernel(x_ref, t_ref, o_ref):
    i = pl.program_id(0)
    x = x_ref[...]
    t = t_ref[...]
    s = jnp.sum(jnp.maximum(x, 0.0) - x * t + jnp.log1p(jnp.exp(-jnp.abs(x))))

    @pl.when(i == 0)
    def _():
        o_ref[0, 0] = s

    @pl.when(i != 0)
    def _():
        o_ref[0, 0] = o_ref[0, 0] + s


def _match_kernel(pred_ref, tbox_ref, anch_ref, tcls_ref, valid_ref, o_ref):
    pred = pred_ref[...]
    tbox = tbox_ref[...]
    anch = anch_ref[...]
    tcls = tcls_ref[...]
    v = valid_ref[...]

    def sig(z):
        return 1.0 / (1.0 + jnp.exp(-z))

    eps = 1e-7
    x1 = sig(pred[0:1, :]) * 2.0 - 0.5
    y1 = sig(pred[1:2, :]) * 2.0 - 0.5
    z1 = sig(pred[2:3, :]) * 2.0 - 0.5
    r1 = (sig(pred[3:4, :]) * 2.0) ** 2 * anch
    x2 = tbox[0:1, :]
    y2 = tbox[1:2, :]
    z2 = tbox[2:3, :]
    r2 = tbox[3:4, :]
    iw = jnp.clip(jnp.minimum(x1 + r1, x2 + r2) - jnp.maximum(x1 - r1, x2 - r2), 0.0, None)
    ih = jnp.clip(jnp.minimum(y1 + r1, y2 + r2) - jnp.maximum(y1 - r1, y2 - r2), 0.0, None)
    idp = jnp.clip(jnp.minimum(z1 + r1, z2 + r2) - jnp.maximum(z1 - r1, z2 - r2), 0.0, None)
    inter = iw * ih * idp
    vol1 = (2.0 * r1) ** 3
    vol2 = (2.0 * r2) ** 3
    union = vol1 + vol2 - inter + eps
    iou = inter / union
    cw = jnp.maximum(x1 + r1, x2 + r2) - jnp.minimum(x1 - r1, x2 - r2)
    ch = jnp.maximum(y1 + r1, y2 + r2) - jnp.minimum(y1 - r1, y2 - r2)
    cd = jnp.maximum(z1 + r1, z2 + r2) - jnp.minimum(z1 - r1, z2 - r2)
    c2 = cw ** 2 + ch ** 2 + cd ** 2 + eps
    rho2 = (x1 - x2) ** 2 + (y1 - y2) ** 2 + (z1 - z2) ** 2
    dr2 = (2.0 * r1 - 2.0 * r2) ** 2
    eiou = iou - rho2 / c2 - dr2 / (cw ** 2 + eps) - dr2 / (ch ** 2 + eps) - dr2 / (cd ** 2 + eps)

    pc = pred[5:15, :]
    elem = jnp.maximum(pc, 0.0) - pc * tcls + jnp.log1p(jnp.exp(-jnp.abs(pc)))

    o_ref[0, 0] = jnp.sum(v)
    o_ref[0, 1] = jnp.sum(v * (1.0 - eiou))
    o_ref[0, 2] = jnp.sum(v * elem)


def kernel(p, targets, anchor):
    B, A, K, J, I, C = p.shape
    indices, tbox, anch, tcls, valid = _prep(p.shape, targets, anchor)
    b_idx, a_idx, gk, gj, gi = indices

    pred = p[b_idx, a_idx, gk, gj, gi]

    bs_sc = jnp.where(valid, b_idx, jnp.int32(B))
    tobj = jnp.zeros(p.shape[:5], jnp.float32).at[bs_sc, a_idx, gk, gj, gi].set(1.0, mode='drop')

    ncell = B * A * K * J * I
    x2d = p[..., 4].reshape(2592, 1024)
    t2d = tobj.reshape(2592, 1024)
    obj_sum = pl.pallas_call(
        _obj_kernel,
        grid=(9,),
        in_specs=[
            pl.BlockSpec((288, 1024), lambda i: (i, 0)),
            pl.BlockSpec((288, 1024), lambda i: (i, 0)),
        ],
        out_specs=pl.BlockSpec((1, 1), lambda i: (0, 0), memory_space=pltpu.SMEM),
        out_shape=jax.ShapeDtypeStruct((1, 1), jnp.float32),
    )(x2d, t2d)

    validf = valid.astype(jnp.float32)[None, :]
    sums = pl.pallas_call(
        _match_kernel,
        out_shape=jax.ShapeDtypeStruct((1, 3), jnp.float32),
        out_specs=pl.BlockSpec(memory_space=pltpu.SMEM),
    )(pred.T, tbox.T, anch.T, tcls.T, validf)

    cnt = sums[0, 0]
    denom = jnp.maximum(cnt, 1.0)
    loss_bbox = (sums[0, 1] / denom).reshape(1) * _BBOX_W
    loss_cls = (sums[0, 2] / (denom * _CLASS_NUM)).reshape(1) * _CLS_W
    loss_obj = (obj_sum[0, 0] / ncell).reshape(1) * _OBJ_W
    total = (loss_bbox + loss_obj + loss_cls) * B
    return (total, loss_obj, loss_cls)
